# Initial kernel scaffold; baseline (speedup 1.0000x reference)
import functools

import jax
import jax.numpy as jnp
from jax import lax
from jax.experimental import pallas as pl
from jax.experimental.pallas import tpu as pltpu
from jax.experimental.pallas import tpu_sc as plsc

N = 50000
IN_DIM = 19
HIDDEN = 64

NC = 2
NS = 16
HALF = 25000
ACC_ROWS = 25088
TRASH = 25080
CHUNK = 128
SUP = 12
SUPE = SUP * CHUNK
WB = 1562
ROWS_PT = ACC_ROWS // NS


def _sc_agg_body(h_hbm, src_hbm, dst_hbm, out_hbm,
                 src_v, dst_v, ldst_v, rows_v, zero_v, acc_sh):
    c = lax.axis_index("c")
    s = lax.axis_index("s")
    base = c * HALF

    def zrow(i, carry):
        for j in range(HIDDEN // 16):
            zero_v[i, pl.ds(j * 16, 16)] = jnp.zeros((16,), jnp.float32)
        return carry
    lax.fori_loop(0, 128, zrow, 0)
    rbase = s * ROWS_PT

    def zsplat(i, carry):
        pltpu.sync_copy(zero_v, acc_sh.at[pl.ds(rbase + i * 128, 128)])
        return carry
    lax.fori_loop(0, 12, zsplat, 0)
    pltpu.sync_copy(zero_v.at[pl.ds(0, 32)],
                    acc_sh.at[pl.ds(rbase + 1536, 32)])
    plsc.subcore_barrier()

    total_rows = src_hbm.shape[0]
    spt = total_rows // (NS * SUP)
    row0 = s * spt * SUP

    def step(t, carry):
        r = row0 + t * SUP
        pltpu.sync_copy(src_hbm.at[pl.ds(r, SUP)], src_v)
        pltpu.sync_copy(dst_hbm.at[pl.ds(r, SUP)], dst_v)
        for j in range(SUP):
            for j2 in range(CHUNK // 16):
                d = dst_v[j, pl.ds(j2 * 16, 16)]
                mine = (d >= base) & (d < base + HALF)
                ldst_v[pl.ds(j2 * 16, 16)] = jnp.where(mine, d - base, TRASH)
            pltpu.sync_copy(h_hbm.at[src_v.at[j]], rows_v)
            pltpu.sync_copy(rows_v, acc_sh.at[ldst_v], add=True)
        return carry
    lax.fori_loop(0, spt, step, 0)
    plsc.subcore_barrier()

    pltpu.sync_copy(acc_sh.at[pl.ds(s * WB, WB)],
                    out_hbm.at[pl.ds(base + s * WB, WB)])

    @pl.when(s == 0)
    def _tail():
        pltpu.sync_copy(acc_sh.at[pl.ds(NS * WB, 8)],
                        out_hbm.at[pl.ds(base + NS * WB, 8)])


@jax.jit
def _sc_agg(h, gsrc, sdst):
    kfn = pl.kernel(
        _sc_agg_body,
        out_type=jax.ShapeDtypeStruct((N, HIDDEN), jnp.float32),
        mesh=plsc.VectorSubcoreMesh(core_axis_name="c", subcore_axis_name="s"),
        scratch_types=[
            pltpu.VMEM((SUP, CHUNK), jnp.int32),
            pltpu.VMEM((SUP, CHUNK), jnp.int32),
            pltpu.VMEM((CHUNK,), jnp.int32),
            pltpu.VMEM((CHUNK, HIDDEN), jnp.float32),
            pltpu.VMEM((128, HIDDEN), jnp.float32),
            pltpu.VMEM_SHARED((ACC_ROWS, HIDDEN), jnp.float32),
        ],
    )
    return kfn(h, gsrc, sdst)


BLK = 2500


def _ln(x, g, b):
    mu = jnp.mean(x, axis=-1, keepdims=True)
    xc = x - mu
    var = jnp.mean(xc * xc, axis=-1, keepdims=True)
    return g * xc * lax.rsqrt(var + 1e-5) + b


def _gin_mlp_body(h_ref, agg_ref, w1_ref, b1_ref, g_ref, bt_ref,
                  w2_ref, b2_ref, dir_ref, lng_ref, lnb_ref, out_ref):
    h = h_ref[...]
    t = h + agg_ref[...]
    u = jnp.dot(t, w1_ref[...], preferred_element_type=jnp.float32) + b1_ref[...]
    u = jnp.maximum(u, 0.0)
    u = _ln(u, g_ref[...], bt_ref[...])
    v = jnp.dot(u, w2_ref[...], preferred_element_type=jnp.float32) + b2_ref[...]
    w = jnp.maximum(v + h + dir_ref[...], 0.0)
    out_ref[...] = _ln(w, lng_ref[...], lnb_ref[...])


def _row_spec(d):
    return pl.BlockSpec((BLK, d), lambda i: (i, 0))


def _full_spec(shape):
    nd = len(shape)
    return pl.BlockSpec(shape, lambda i: (0,) * nd)


@jax.jit
def _tc_gin_mlp(h, agg, w1, b1, g, bt, w2, b2, dirv, lng, lnb):
    return pl.pallas_call(
        _gin_mlp_body,
        grid=(N // BLK,),
        in_specs=[
            _row_spec(HIDDEN), _row_spec(HIDDEN),
            _full_spec((HIDDEN, HIDDEN)), _full_spec((1, HIDDEN)),
            _full_spec((1, HIDDEN)), _full_spec((1, HIDDEN)),
            _full_spec((HIDDEN, HIDDEN)), _full_spec((1, HIDDEN)),
            _full_spec((1, HIDDEN)), _full_spec((1, HIDDEN)),
            _full_spec((1, HIDDEN)),
        ],
        out_specs=_row_spec(HIDDEN),
        out_shape=jax.ShapeDtypeStruct((N, HIDDEN), jnp.float32),
    )(h, agg, w1, b1, g, bt, w2, b2, dirv, lng, lnb)


def _encode_body(x_ref, flag_ref, w_ref, wflag_ref, b_ref, out_ref):
    out_ref[...] = (
        jnp.dot(x_ref[...], w_ref[...], preferred_element_type=jnp.float32)
        + flag_ref[...] * wflag_ref[...] + b_ref[...])


@jax.jit
def _tc_encode(x_m, flag, w_x, w_flag, bias):
    return pl.pallas_call(
        _encode_body,
        grid=(N // BLK,),
        in_specs=[
            _row_spec(IN_DIM), _row_spec(1),
            _full_spec((IN_DIM, HIDDEN)), _full_spec((1, HIDDEN)),
            _full_spec((1, HIDDEN)),
        ],
        out_specs=_row_spec(HIDDEN),
        out_shape=jax.ShapeDtypeStruct((N, HIDDEN), jnp.float32),
    )(x_m, flag, w_x, w_flag, bias)


def _out_body(h_ref, w1_ref, b1_ref, g_ref, bt_ref, w2_ref, b2_ref, out_ref):
    u = jnp.dot(h_ref[...], w1_ref[...], preferred_element_type=jnp.float32)
    u = jnp.maximum(u + b1_ref[...], 0.0)
    u = _ln(u, g_ref[...], bt_ref[...])
    out_ref[...] = (
        jnp.dot(u, w2_ref[...], preferred_element_type=jnp.float32)
        + b2_ref[...])


@jax.jit
def _tc_out(h, w1, b1, g, bt, w2, b2):
    return pl.pallas_call(
        _out_body,
        grid=(N // BLK,),
        in_specs=[
            _row_spec(HIDDEN),
            _full_spec((HIDDEN, HIDDEN)), _full_spec((1, HIDDEN)),
            _full_spec((1, HIDDEN)), _full_spec((1, HIDDEN)),
            _full_spec((HIDDEN, IN_DIM)), _full_spec((1, IN_DIM)),
        ],
        out_specs=_row_spec(IN_DIM),
        out_shape=jax.ShapeDtypeStruct((N, IN_DIM), jnp.float32),
    )(h, w1, b1, g, bt, w2, b2)


def _r(v):
    return v.reshape(1, -1)


def _down_up(h, gsrc_f, sdst_f, gsrc_r, sdst_r, lp):
    p = lp["down"]
    agg = _sc_agg(h, gsrc_f, sdst_f)
    h = _tc_gin_mlp(h, agg, p["W1"], _r(p["b1"]), _r(p["g"]), _r(p["bt"]),
                    p["W2"], _r(p["b2"]), _r(lp["dir"][0]),
                    _r(lp["ln1g"]), _r(lp["ln1b"]))
    p = lp["up"]
    agg = _sc_agg(h, gsrc_r, sdst_r)
    h = _tc_gin_mlp(h, agg, p["W1"], _r(p["b1"]), _r(p["g"]), _r(p["bt"]),
                    p["W2"], _r(p["b2"]), _r(lp["dir"][1]),
                    _r(lp["ln2g"]), _r(lp["ln2b"]))
    return h


def kernel(x, edge_index, root_index, mask_idx, params):
    e = edge_index.shape[1]
    supe_all = NS * SUPE
    e_pad = ((e + supe_all - 1) // supe_all) * supe_all
    pad = e_pad - e
    src = edge_index[0]
    dst = edge_index[1]
    zpad = jnp.zeros((pad,), jnp.int32)
    npad = jnp.full((pad,), -1, jnp.int32)
    gsrc_f = jnp.concatenate([src, zpad]).reshape(-1, CHUNK)
    sdst_f = jnp.concatenate([dst, npad]).reshape(-1, CHUNK)
    gsrc_r = jnp.concatenate([dst, zpad]).reshape(-1, CHUNK)
    sdst_r = jnp.concatenate([src, npad]).reshape(-1, CHUNK)

    flag = jnp.zeros((N, 1), x.dtype).at[mask_idx].set(1.0)
    x_m = x.at[mask_idx].set(0.0)

    p = params
    h = _tc_encode(x_m, flag, p["enc_in_W"][:IN_DIM],
                   _r(p["enc_in_W"][IN_DIM]), _r(p["enc_in_b"]))
    for lp in p["enc_layers"]:
        h = _down_up(h, gsrc_f, sdst_f, gsrc_r, sdst_r, lp)

    z = jnp.take(h, root_index, axis=0)
    dec_bias = z @ p["dec_in_W"][IN_DIM + 1:] + _r(p["dec_in_b"])
    hd = _tc_encode(x_m, flag, p["dec_in_W"][:IN_DIM],
                    _r(p["dec_in_W"][IN_DIM]), dec_bias)
    for lp in p["dec_layers"]:
        hd = _down_up(hd, gsrc_f, sdst_f, gsrc_r, sdst_r, lp)

    return _tc_out(hd, p["out_W1"], _r(p["out_b1"]), _r(p["out_g"]),
                   _r(p["out_bt"]), p["out_W2"], _r(p["out_b2"]))


# speedup vs baseline: 2.4783x; 2.4783x over previous
import functools

import jax
import jax.numpy as jnp
from jax import lax
from jax.experimental import pallas as pl
from jax.experimental.pallas import tpu as pltpu
from jax.experimental.pallas import tpu_sc as plsc

N = 50000
IN_DIM = 19
HIDDEN = 64

NC = 2
NS = 16
HALF = 25000
ACC_ROWS = 25088
TRASH = 25080
CHUNK = 128
SUP = 16
SUPE = SUP * CHUNK
WB = 1560
ROWS_PT = ACC_ROWS // NS


def _sc_agg_body(h_hbm, src_hbm, dst_hbm, out_hbm,
                 src_v, dst_v, ldst_v, rows_v, zero_v, acc_sh):
    c = lax.axis_index("c")
    s = lax.axis_index("s")
    base = c * HALF

    def zrow(i, carry):
        for j in range(HIDDEN // 16):
            zero_v[i, pl.ds(j * 16, 16)] = jnp.zeros((16,), jnp.float32)
        return carry
    lax.fori_loop(0, 128, zrow, 0)
    rbase = s * ROWS_PT

    def zsplat(i, carry):
        pltpu.sync_copy(zero_v, acc_sh.at[pl.ds(rbase + i * 128, 128)])
        return carry
    lax.fori_loop(0, 12, zsplat, 0)
    pltpu.sync_copy(zero_v.at[pl.ds(0, 32)],
                    acc_sh.at[pl.ds(rbase + 1536, 32)])
    plsc.subcore_barrier()

    total_rows = src_hbm.shape[0]
    spt = total_rows // (NS * SUP)
    row0 = s * spt * SUP

    def step(t, carry):
        r = row0 + t * SUP
        pltpu.sync_copy(src_hbm.at[pl.ds(r, SUP)], src_v)
        pltpu.sync_copy(dst_hbm.at[pl.ds(r, SUP)], dst_v)
        for j in range(SUP):
            for j2 in range(CHUNK // 16):
                d = dst_v[j, pl.ds(j2 * 16, 16)]
                mine = (d >= base) & (d < base + HALF)
                ldst_v[pl.ds(j2 * 16, 16)] = jnp.where(mine, d - base, TRASH)
            pltpu.sync_copy(h_hbm.at[src_v.at[j]], rows_v)
            pltpu.sync_copy(rows_v, acc_sh.at[ldst_v], add=True)
        return carry
    lax.fori_loop(0, spt, step, 0)
    plsc.subcore_barrier()

    pltpu.sync_copy(acc_sh.at[pl.ds(s * WB, WB)],
                    out_hbm.at[pl.ds(base + s * WB, WB)])

    @pl.when(s == 0)
    def _tail():
        pltpu.sync_copy(acc_sh.at[pl.ds(NS * WB, HALF - NS * WB)],
                        out_hbm.at[pl.ds(base + NS * WB, HALF - NS * WB)])


@jax.jit
def _sc_agg(h, gsrc, sdst):
    kfn = pl.kernel(
        _sc_agg_body,
        out_type=jax.ShapeDtypeStruct((N, HIDDEN), jnp.float32),
        mesh=plsc.VectorSubcoreMesh(core_axis_name="c", subcore_axis_name="s"),
        compiler_params=pltpu.CompilerParams(use_tc_tiling_on_sc=False),
        scratch_types=[
            pltpu.VMEM((SUP, CHUNK), jnp.int32),
            pltpu.VMEM((SUP, CHUNK), jnp.int32),
            pltpu.VMEM((CHUNK,), jnp.int32),
            pltpu.VMEM((CHUNK, HIDDEN), jnp.float32),
            pltpu.VMEM((128, HIDDEN), jnp.float32),
            pltpu.VMEM_SHARED((ACC_ROWS, HIDDEN), jnp.float32),
        ],
    )
    return kfn(h, gsrc, sdst)


BLK = 2000


def _ln(x, g, b):
    mu = jnp.mean(x, axis=-1, keepdims=True)
    xc = x - mu
    var = jnp.mean(xc * xc, axis=-1, keepdims=True)
    return g * xc * lax.rsqrt(var + 1e-5) + b


def _gin_mlp_body(h_ref, agg_ref, w1_ref, b1_ref, g_ref, bt_ref,
                  w2_ref, b2_ref, dir_ref, lng_ref, lnb_ref, out_ref):
    h = h_ref[...]
    t = h + agg_ref[...]
    u = jnp.dot(t, w1_ref[...], preferred_element_type=jnp.float32) + b1_ref[...]
    u = jnp.maximum(u, 0.0)
    u = _ln(u, g_ref[...], bt_ref[...])
    v = jnp.dot(u, w2_ref[...], preferred_element_type=jnp.float32) + b2_ref[...]
    w = jnp.maximum(v + h + dir_ref[...], 0.0)
    out_ref[...] = _ln(w, lng_ref[...], lnb_ref[...])


def _row_spec(d):
    return pl.BlockSpec((BLK, d), lambda i: (i, 0))


def _full_spec(shape):
    nd = len(shape)
    return pl.BlockSpec(shape, lambda i: (0,) * nd)


@jax.jit
def _tc_gin_mlp(h, agg, w1, b1, g, bt, w2, b2, dirv, lng, lnb):
    return pl.pallas_call(
        _gin_mlp_body,
        grid=(N // BLK,),
        in_specs=[
            _row_spec(HIDDEN), _row_spec(HIDDEN),
            _full_spec((HIDDEN, HIDDEN)), _full_spec((1, HIDDEN)),
            _full_spec((1, HIDDEN)), _full_spec((1, HIDDEN)),
            _full_spec((HIDDEN, HIDDEN)), _full_spec((1, HIDDEN)),
            _full_spec((1, HIDDEN)), _full_spec((1, HIDDEN)),
            _full_spec((1, HIDDEN)),
        ],
        out_specs=_row_spec(HIDDEN),
        out_shape=jax.ShapeDtypeStruct((N, HIDDEN), jnp.float32),
    )(h, agg, w1, b1, g, bt, w2, b2, dirv, lng, lnb)


def _encode_body(x_ref, flag_ref, w_ref, wflag_ref, b_ref, out_ref):
    out_ref[...] = (
        jnp.dot(x_ref[...], w_ref[...], preferred_element_type=jnp.float32)
        + flag_ref[...] * wflag_ref[...] + b_ref[...])


@jax.jit
def _tc_encode(x_m, flag, w_x, w_flag, bias):
    return pl.pallas_call(
        _encode_body,
        grid=(N // BLK,),
        in_specs=[
            _row_spec(IN_DIM), _row_spec(1),
            _full_spec((IN_DIM, HIDDEN)), _full_spec((1, HIDDEN)),
            _full_spec((1, HIDDEN)),
        ],
        out_specs=_row_spec(HIDDEN),
        out_shape=jax.ShapeDtypeStruct((N, HIDDEN), jnp.float32),
    )(x_m, flag, w_x, w_flag, bias)


def _out_body(h_ref, w1_ref, b1_ref, g_ref, bt_ref, w2_ref, b2_ref, out_ref):
    u = jnp.dot(h_ref[...], w1_ref[...], preferred_element_type=jnp.float32)
    u = jnp.maximum(u + b1_ref[...], 0.0)
    u = _ln(u, g_ref[...], bt_ref[...])
    out_ref[...] = (
        jnp.dot(u, w2_ref[...], preferred_element_type=jnp.float32)
        + b2_ref[...])


@jax.jit
def _tc_out(h, w1, b1, g, bt, w2, b2):
    return pl.pallas_call(
        _out_body,
        grid=(N // BLK,),
        in_specs=[
            _row_spec(HIDDEN),
            _full_spec((HIDDEN, HIDDEN)), _full_spec((1, HIDDEN)),
            _full_spec((1, HIDDEN)), _full_spec((1, HIDDEN)),
            _full_spec((HIDDEN, IN_DIM)), _full_spec((1, IN_DIM)),
        ],
        out_specs=_row_spec(IN_DIM),
        out_shape=jax.ShapeDtypeStruct((N, IN_DIM), jnp.float32),
    )(h, w1, b1, g, bt, w2, b2)


def _r(v):
    return v.reshape(1, -1)


def _down_up(h, gsrc_f, sdst_f, gsrc_r, sdst_r, lp):
    p = lp["down"]
    agg = _sc_agg(h, gsrc_f, sdst_f)
    h = _tc_gin_mlp(h, agg, p["W1"], _r(p["b1"]), _r(p["g"]), _r(p["bt"]),
                    p["W2"], _r(p["b2"]), _r(lp["dir"][0]),
                    _r(lp["ln1g"]), _r(lp["ln1b"]))
    p = lp["up"]
    agg = _sc_agg(h, gsrc_r, sdst_r)
    h = _tc_gin_mlp(h, agg, p["W1"], _r(p["b1"]), _r(p["g"]), _r(p["bt"]),
                    p["W2"], _r(p["b2"]), _r(lp["dir"][1]),
                    _r(lp["ln2g"]), _r(lp["ln2b"]))
    return h


def kernel(x, edge_index, root_index, mask_idx, params):
    e = edge_index.shape[1]
    supe_all = NS * SUPE
    e_pad = ((e + supe_all - 1) // supe_all) * supe_all
    pad = e_pad - e
    src = edge_index[0]
    dst = edge_index[1]
    zpad = jnp.zeros((pad,), jnp.int32)
    npad = jnp.full((pad,), -1, jnp.int32)
    gsrc_f = jnp.concatenate([src, zpad]).reshape(-1, CHUNK)
    sdst_f = jnp.concatenate([dst, npad]).reshape(-1, CHUNK)
    gsrc_r = jnp.concatenate([dst, zpad]).reshape(-1, CHUNK)
    sdst_r = jnp.concatenate([src, npad]).reshape(-1, CHUNK)

    flag = jnp.zeros((N, 1), x.dtype).at[mask_idx].set(1.0)
    x_m = x.at[mask_idx].set(0.0)

    p = params
    h = _tc_encode(x_m, flag, p["enc_in_W"][:IN_DIM],
                   _r(p["enc_in_W"][IN_DIM]), _r(p["enc_in_b"]))
    for lp in p["enc_layers"]:
        h = _down_up(h, gsrc_f, sdst_f, gsrc_r, sdst_r, lp)

    z = jnp.take(h, root_index, axis=0)
    dec_bias = z @ p["dec_in_W"][IN_DIM + 1:] + _r(p["dec_in_b"])
    hd = _tc_encode(x_m, flag, p["dec_in_W"][:IN_DIM],
                    _r(p["dec_in_W"][IN_DIM]), dec_bias)
    for lp in p["dec_layers"]:
        hd = _down_up(hd, gsrc_f, sdst_f, gsrc_r, sdst_r, lp)

    return _tc_out(hd, p["out_W1"], _r(p["out_b1"]), _r(p["out_g"]),
                   _r(p["out_bt"]), p["out_W2"], _r(p["out_b2"]))


# speedup vs baseline: 4.3645x; 1.7611x over previous
import functools

import jax
import jax.numpy as jnp
from jax import lax
from jax.experimental import pallas as pl
from jax.experimental.pallas import tpu as pltpu
from jax.experimental.pallas import tpu_sc as plsc

N = 50000
IN_DIM = 19
HIDDEN = 64

NC = 2
NS = 16
HALF = 25000
ACC_ROWS = 25088
TRASH = 25080
CHUNK = 128
SUP = 8
SUPE = SUP * CHUNK
T = 49
WB = 1560
ROWS_PT = ACC_ROWS // NS


def _sc_agg_body(h_hbm, src_hbm, dst_hbm, out_hbm,
                 srcv, dstv, ld_a, ld_b, r_a, r_b,
                 acc_sh, gsa, gsb, ssa, ssb):
    c = lax.axis_index("c")
    s = lax.axis_index("s")
    base = c * HALF

    def zrow(i, carry):
        for j in range(HIDDEN // 16):
            r_a[i, pl.ds(j * 16, 16)] = jnp.zeros((16,), jnp.float32)
        return carry
    lax.fori_loop(0, 128, zrow, 0)
    rbase = s * ROWS_PT

    def zsplat(i, carry):
        pltpu.sync_copy(r_a, acc_sh.at[pl.ds(rbase + i * 128, 128)])
        return carry
    lax.fori_loop(0, 12, zsplat, 0)
    pltpu.sync_copy(r_a.at[pl.ds(0, 32)], acc_sh.at[pl.ds(rbase + 1536, 32)])
    plsc.subcore_barrier()

    row0 = s * T * SUP

    def masks(ld, j):
        for k in range(CHUNK // 16):
            d = dstv[j, pl.ds(k * 16, 16)]
            mine = (d >= base) & (d < base + HALF)
            ld[pl.ds(k * 16, 16)] = jnp.where(mine, d - base, TRASH)

    def drain(R, sem):
        pltpu.make_async_copy(h_hbm.at[pl.ds(0, CHUNK)], R, sem).wait()

    def step(t, carry):
        r = row0 + t * SUP
        pltpu.sync_copy(src_hbm.at[pl.ds(r, SUP)], srcv)
        pltpu.sync_copy(dst_hbm.at[pl.ds(r, SUP)], dstv)
        for i in range(0, SUP, 2):
            if i == 0:
                @pl.when(t > 0)
                def _():
                    drain(r_a, ssa)
            else:
                drain(r_a, ssa)
            masks(ld_a, i)
            pltpu.async_copy(h_hbm.at[srcv.at[i]], r_a, gsa)
            if i == 0:
                @pl.when(t > 0)
                def _():
                    drain(r_b, ssb)
            else:
                drain(r_b, ssb)
            masks(ld_b, i + 1)
            pltpu.async_copy(h_hbm.at[srcv.at[i + 1]], r_b, gsb)
            drain(r_a, gsa)
            pltpu.async_copy(r_a, acc_sh.at[ld_a], ssa, add=True)
            drain(r_b, gsb)
            pltpu.async_copy(r_b, acc_sh.at[ld_b], ssb, add=True)
        return carry
    lax.fori_loop(0, T, step, 0)
    drain(r_a, ssa)
    drain(r_b, ssb)
    plsc.subcore_barrier()

    pltpu.sync_copy(acc_sh.at[pl.ds(s * WB, WB)],
                    out_hbm.at[pl.ds(base + s * WB, WB)])

    @pl.when(s == 0)
    def _tail():
        pltpu.sync_copy(acc_sh.at[pl.ds(NS * WB, HALF - NS * WB)],
                        out_hbm.at[pl.ds(base + NS * WB, HALF - NS * WB)])


@jax.jit
def _sc_agg(h, gsrc, sdst):
    kfn = pl.kernel(
        _sc_agg_body,
        out_type=jax.ShapeDtypeStruct((N, HIDDEN), jnp.float32),
        mesh=plsc.VectorSubcoreMesh(core_axis_name="c", subcore_axis_name="s"),
        compiler_params=pltpu.CompilerParams(use_tc_tiling_on_sc=False),
        scratch_types=(
            [pltpu.VMEM((SUP, CHUNK), jnp.int32)] * 2
            + [pltpu.VMEM((CHUNK,), jnp.int32)] * 2
            + [pltpu.VMEM((CHUNK, HIDDEN), jnp.float32)] * 2
            + [pltpu.VMEM_SHARED((ACC_ROWS, HIDDEN), jnp.float32)]
            + [pltpu.SemaphoreType.DMA] * 4
        ),
    )
    return kfn(h, gsrc, sdst)


BLK = 2000


def _ln(x, g, b):
    mu = jnp.mean(x, axis=-1, keepdims=True)
    xc = x - mu
    var = jnp.mean(xc * xc, axis=-1, keepdims=True)
    return g * xc * lax.rsqrt(var + 1e-5) + b


def _gin_mlp_body(h_ref, agg_ref, w1_ref, b1_ref, g_ref, bt_ref,
                  w2_ref, b2_ref, dir_ref, lng_ref, lnb_ref, out_ref):
    h = h_ref[...]
    t = h + agg_ref[...]
    u = jnp.dot(t, w1_ref[...], preferred_element_type=jnp.float32) + b1_ref[...]
    u = jnp.maximum(u, 0.0)
    u = _ln(u, g_ref[...], bt_ref[...])
    v = jnp.dot(u, w2_ref[...], preferred_element_type=jnp.float32) + b2_ref[...]
    w = jnp.maximum(v + h + dir_ref[...], 0.0)
    out_ref[...] = _ln(w, lng_ref[...], lnb_ref[...])


def _row_spec(d):
    return pl.BlockSpec((BLK, d), lambda i: (i, 0))


def _full_spec(shape):
    nd = len(shape)
    return pl.BlockSpec(shape, lambda i: (0,) * nd)


@jax.jit
def _tc_gin_mlp(h, agg, w1, b1, g, bt, w2, b2, dirv, lng, lnb):
    return pl.pallas_call(
        _gin_mlp_body,
        grid=(N // BLK,),
        in_specs=[
            _row_spec(HIDDEN), _row_spec(HIDDEN),
            _full_spec((HIDDEN, HIDDEN)), _full_spec((1, HIDDEN)),
            _full_spec((1, HIDDEN)), _full_spec((1, HIDDEN)),
            _full_spec((HIDDEN, HIDDEN)), _full_spec((1, HIDDEN)),
            _full_spec((1, HIDDEN)), _full_spec((1, HIDDEN)),
            _full_spec((1, HIDDEN)),
        ],
        out_specs=_row_spec(HIDDEN),
        out_shape=jax.ShapeDtypeStruct((N, HIDDEN), jnp.float32),
    )(h, agg, w1, b1, g, bt, w2, b2, dirv, lng, lnb)


def _encode_body(x_ref, flag_ref, w_ref, wflag_ref, b_ref, out_ref):
    out_ref[...] = (
        jnp.dot(x_ref[...], w_ref[...], preferred_element_type=jnp.float32)
        + flag_ref[...] * wflag_ref[...] + b_ref[...])


@jax.jit
def _tc_encode(x_m, flag, w_x, w_flag, bias):
    return pl.pallas_call(
        _encode_body,
        grid=(N // BLK,),
        in_specs=[
            _row_spec(IN_DIM), _row_spec(1),
            _full_spec((IN_DIM, HIDDEN)), _full_spec((1, HIDDEN)),
            _full_spec((1, HIDDEN)),
        ],
        out_specs=_row_spec(HIDDEN),
        out_shape=jax.ShapeDtypeStruct((N, HIDDEN), jnp.float32),
    )(x_m, flag, w_x, w_flag, bias)


def _out_body(h_ref, w1_ref, b1_ref, g_ref, bt_ref, w2_ref, b2_ref, out_ref):
    u = jnp.dot(h_ref[...], w1_ref[...], preferred_element_type=jnp.float32)
    u = jnp.maximum(u + b1_ref[...], 0.0)
    u = _ln(u, g_ref[...], bt_ref[...])
    out_ref[...] = (
        jnp.dot(u, w2_ref[...], preferred_element_type=jnp.float32)
        + b2_ref[...])


@jax.jit
def _tc_out(h, w1, b1, g, bt, w2, b2):
    return pl.pallas_call(
        _out_body,
        grid=(N // BLK,),
        in_specs=[
            _row_spec(HIDDEN),
            _full_spec((HIDDEN, HIDDEN)), _full_spec((1, HIDDEN)),
            _full_spec((1, HIDDEN)), _full_spec((1, HIDDEN)),
            _full_spec((HIDDEN, IN_DIM)), _full_spec((1, IN_DIM)),
        ],
        out_specs=_row_spec(IN_DIM),
        out_shape=jax.ShapeDtypeStruct((N, IN_DIM), jnp.float32),
    )(h, w1, b1, g, bt, w2, b2)


def _r(v):
    return v.reshape(1, -1)


def _down_up(h, gsrc_f, sdst_f, gsrc_r, sdst_r, lp):
    p = lp["down"]
    agg = _sc_agg(h, gsrc_f, sdst_f)
    h = _tc_gin_mlp(h, agg, p["W1"], _r(p["b1"]), _r(p["g"]), _r(p["bt"]),
                    p["W2"], _r(p["b2"]), _r(lp["dir"][0]),
                    _r(lp["ln1g"]), _r(lp["ln1b"]))
    p = lp["up"]
    agg = _sc_agg(h, gsrc_r, sdst_r)
    h = _tc_gin_mlp(h, agg, p["W1"], _r(p["b1"]), _r(p["g"]), _r(p["bt"]),
                    p["W2"], _r(p["b2"]), _r(lp["dir"][1]),
                    _r(lp["ln2g"]), _r(lp["ln2b"]))
    return h


def kernel(x, edge_index, root_index, mask_idx, params):
    e = edge_index.shape[1]
    e_pad = NS * T * SUPE
    pad = e_pad - e
    src = edge_index[0]
    dst = edge_index[1]
    zpad = jnp.zeros((pad,), jnp.int32)
    npad = jnp.full((pad,), -1, jnp.int32)
    gsrc_f = jnp.concatenate([src, zpad]).reshape(-1, CHUNK)
    sdst_f = jnp.concatenate([dst, npad]).reshape(-1, CHUNK)
    gsrc_r = jnp.concatenate([dst, zpad]).reshape(-1, CHUNK)
    sdst_r = jnp.concatenate([src, npad]).reshape(-1, CHUNK)

    flag = jnp.zeros((N, 1), x.dtype).at[mask_idx].set(1.0)
    x_m = x.at[mask_idx].set(0.0)

    p = params
    h = _tc_encode(x_m, flag, p["enc_in_W"][:IN_DIM],
                   _r(p["enc_in_W"][IN_DIM]), _r(p["enc_in_b"]))
    for lp in p["enc_layers"]:
        h = _down_up(h, gsrc_f, sdst_f, gsrc_r, sdst_r, lp)

    z = jnp.take(h, root_index, axis=0)
    dec_bias = z @ p["dec_in_W"][IN_DIM + 1:] + _r(p["dec_in_b"])
    hd = _tc_encode(x_m, flag, p["dec_in_W"][:IN_DIM],
                    _r(p["dec_in_W"][IN_DIM]), dec_bias)
    for lp in p["dec_layers"]:
        hd = _down_up(hd, gsrc_f, sdst_f, gsrc_r, sdst_r, lp)

    return _tc_out(hd, p["out_W1"], _r(p["out_b1"]), _r(p["out_g"]),
                   _r(p["out_bt"]), p["out_W2"], _r(p["out_b2"]))


# speedup vs baseline: 5.1200x; 1.1731x over previous
import functools

import jax
import jax.numpy as jnp
from jax import lax
from jax.experimental import pallas as pl
from jax.experimental.pallas import tpu as pltpu
from jax.experimental.pallas import tpu_sc as plsc

N = 50000
IN_DIM = 19
HIDDEN = 64

NC = 2
NS = 16
HALF = 25000
ACC_ROWS = 25088
TRASH = 25080
CHUNK = 128
SUP = 8
SUPE = SUP * CHUNK
T = 49
WB = 1560
ROWS_PT = ACC_ROWS // NS


def _sc_agg_body(h_hbm, src_hbm, dst_hbm, out_hbm,
                 srcv, dstv, ld_a, ld_b, r_a, r_b,
                 acc_sh, gsa, gsb, ssa, ssb):
    c = lax.axis_index("c")
    s = lax.axis_index("s")
    base = c * HALF

    def zrow(i, carry):
        for j in range(HIDDEN // 16):
            r_a[i, pl.ds(j * 16, 16)] = jnp.zeros((16,), jnp.float32)
        return carry
    lax.fori_loop(0, 128, zrow, 0)
    rbase = s * ROWS_PT

    def zsplat(i, carry):
        pltpu.sync_copy(r_a, acc_sh.at[pl.ds(rbase + i * 128, 128)])
        return carry
    lax.fori_loop(0, 12, zsplat, 0)
    pltpu.sync_copy(r_a.at[pl.ds(0, 32)], acc_sh.at[pl.ds(rbase + 1536, 32)])
    plsc.subcore_barrier()

    row0 = s * T * SUP

    trash = HALF + s

    def masks(ld, j):
        for k in range(CHUNK // 16):
            d = dstv[j, pl.ds(k * 16, 16)]
            mine = (d >= base) & (d < base + HALF)
            ld[pl.ds(k * 16, 16)] = jnp.where(mine, d - base, trash)

    def drain(R, sem):
        pltpu.make_async_copy(h_hbm.at[pl.ds(0, CHUNK)], R, sem).wait()

    def step(t, carry):
        r = row0 + t * SUP
        pltpu.sync_copy(src_hbm.at[pl.ds(r, SUP)], srcv)
        pltpu.sync_copy(dst_hbm.at[pl.ds(r, SUP)], dstv)
        for i in range(0, SUP, 2):
            if i == 0:
                @pl.when(t > 0)
                def _():
                    drain(r_a, ssa)
            else:
                drain(r_a, ssa)
            masks(ld_a, i)
            pltpu.async_copy(h_hbm.at[srcv.at[i]], r_a, gsa)
            if i == 0:
                @pl.when(t > 0)
                def _():
                    drain(r_b, ssb)
            else:
                drain(r_b, ssb)
            masks(ld_b, i + 1)
            pltpu.async_copy(h_hbm.at[srcv.at[i + 1]], r_b, gsb)
            drain(r_a, gsa)
            pltpu.async_copy(r_a, acc_sh.at[ld_a], ssa, add=True)
            drain(r_b, gsb)
            pltpu.async_copy(r_b, acc_sh.at[ld_b], ssb, add=True)
        return carry
    lax.fori_loop(0, T, step, 0)
    drain(r_a, ssa)
    drain(r_b, ssb)
    plsc.subcore_barrier()

    pltpu.sync_copy(acc_sh.at[pl.ds(s * WB, WB)],
                    out_hbm.at[pl.ds(base + s * WB, WB)])

    @pl.when(s == 0)
    def _tail():
        pltpu.sync_copy(acc_sh.at[pl.ds(NS * WB, HALF - NS * WB)],
                        out_hbm.at[pl.ds(base + NS * WB, HALF - NS * WB)])


@jax.jit
def _sc_agg(h, gsrc, sdst):
    kfn = pl.kernel(
        _sc_agg_body,
        out_type=jax.ShapeDtypeStruct((N, HIDDEN), jnp.float32),
        mesh=plsc.VectorSubcoreMesh(core_axis_name="c", subcore_axis_name="s"),
        compiler_params=pltpu.CompilerParams(use_tc_tiling_on_sc=False),
        scratch_types=(
            [pltpu.VMEM((SUP, CHUNK), jnp.int32)] * 2
            + [pltpu.VMEM((CHUNK,), jnp.int32)] * 2
            + [pltpu.VMEM((CHUNK, HIDDEN), jnp.float32)] * 2
            + [pltpu.VMEM_SHARED((ACC_ROWS, HIDDEN), jnp.float32)]
            + [pltpu.SemaphoreType.DMA] * 4
        ),
    )
    return kfn(h, gsrc, sdst)


BLK = 2000


def _ln(x, g, b):
    mu = jnp.mean(x, axis=-1, keepdims=True)
    xc = x - mu
    var = jnp.mean(xc * xc, axis=-1, keepdims=True)
    return g * xc * lax.rsqrt(var + 1e-5) + b


def _gin_mlp_body(h_ref, agg_ref, w1_ref, b1_ref, g_ref, bt_ref,
                  w2_ref, b2_ref, dir_ref, lng_ref, lnb_ref, out_ref):
    h = h_ref[...]
    t = h + agg_ref[...]
    u = jnp.dot(t, w1_ref[...], preferred_element_type=jnp.float32) + b1_ref[...]
    u = jnp.maximum(u, 0.0)
    u = _ln(u, g_ref[...], bt_ref[...])
    v = jnp.dot(u, w2_ref[...], preferred_element_type=jnp.float32) + b2_ref[...]
    w = jnp.maximum(v + h + dir_ref[...], 0.0)
    out_ref[...] = _ln(w, lng_ref[...], lnb_ref[...])


def _row_spec(d):
    return pl.BlockSpec((BLK, d), lambda i: (i, 0))


def _full_spec(shape):
    nd = len(shape)
    return pl.BlockSpec(shape, lambda i: (0,) * nd)


@jax.jit
def _tc_gin_mlp(h, agg, w1, b1, g, bt, w2, b2, dirv, lng, lnb):
    return pl.pallas_call(
        _gin_mlp_body,
        grid=(N // BLK,),
        in_specs=[
            _row_spec(HIDDEN), _row_spec(HIDDEN),
            _full_spec((HIDDEN, HIDDEN)), _full_spec((1, HIDDEN)),
            _full_spec((1, HIDDEN)), _full_spec((1, HIDDEN)),
            _full_spec((HIDDEN, HIDDEN)), _full_spec((1, HIDDEN)),
            _full_spec((1, HIDDEN)), _full_spec((1, HIDDEN)),
            _full_spec((1, HIDDEN)),
        ],
        out_specs=_row_spec(HIDDEN),
        out_shape=jax.ShapeDtypeStruct((N, HIDDEN), jnp.float32),
    )(h, agg, w1, b1, g, bt, w2, b2, dirv, lng, lnb)


def _encode_body(x_ref, flag_ref, w_ref, wflag_ref, b_ref, out_ref):
    out_ref[...] = (
        jnp.dot(x_ref[...], w_ref[...], preferred_element_type=jnp.float32)
        + flag_ref[...] * wflag_ref[...] + b_ref[...])


@jax.jit
def _tc_encode(x_m, flag, w_x, w_flag, bias):
    return pl.pallas_call(
        _encode_body,
        grid=(N // BLK,),
        in_specs=[
            _row_spec(IN_DIM), _row_spec(1),
            _full_spec((IN_DIM, HIDDEN)), _full_spec((1, HIDDEN)),
            _full_spec((1, HIDDEN)),
        ],
        out_specs=_row_spec(HIDDEN),
        out_shape=jax.ShapeDtypeStruct((N, HIDDEN), jnp.float32),
    )(x_m, flag, w_x, w_flag, bias)


def _out_body(h_ref, w1_ref, b1_ref, g_ref, bt_ref, w2_ref, b2_ref, out_ref):
    u = jnp.dot(h_ref[...], w1_ref[...], preferred_element_type=jnp.float32)
    u = jnp.maximum(u + b1_ref[...], 0.0)
    u = _ln(u, g_ref[...], bt_ref[...])
    out_ref[...] = (
        jnp.dot(u, w2_ref[...], preferred_element_type=jnp.float32)
        + b2_ref[...])


@jax.jit
def _tc_out(h, w1, b1, g, bt, w2, b2):
    return pl.pallas_call(
        _out_body,
        grid=(N // BLK,),
        in_specs=[
            _row_spec(HIDDEN),
            _full_spec((HIDDEN, HIDDEN)), _full_spec((1, HIDDEN)),
            _full_spec((1, HIDDEN)), _full_spec((1, HIDDEN)),
            _full_spec((HIDDEN, IN_DIM)), _full_spec((1, IN_DIM)),
        ],
        out_specs=_row_spec(IN_DIM),
        out_shape=jax.ShapeDtypeStruct((N, IN_DIM), jnp.float32),
    )(h, w1, b1, g, bt, w2, b2)


def _r(v):
    return v.reshape(1, -1)


def _down_up(h, gsrc_f, sdst_f, gsrc_r, sdst_r, lp):
    p = lp["down"]
    agg = _sc_agg(h, gsrc_f, sdst_f)
    h = _tc_gin_mlp(h, agg, p["W1"], _r(p["b1"]), _r(p["g"]), _r(p["bt"]),
                    p["W2"], _r(p["b2"]), _r(lp["dir"][0]),
                    _r(lp["ln1g"]), _r(lp["ln1b"]))
    p = lp["up"]
    agg = _sc_agg(h, gsrc_r, sdst_r)
    h = _tc_gin_mlp(h, agg, p["W1"], _r(p["b1"]), _r(p["g"]), _r(p["bt"]),
                    p["W2"], _r(p["b2"]), _r(lp["dir"][1]),
                    _r(lp["ln2g"]), _r(lp["ln2b"]))
    return h


def kernel(x, edge_index, root_index, mask_idx, params):
    e = edge_index.shape[1]
    e_pad = NS * T * SUPE
    pad = e_pad - e
    src = edge_index[0]
    dst = edge_index[1]
    zpad = jnp.zeros((pad,), jnp.int32)
    npad = jnp.full((pad,), -1, jnp.int32)
    gsrc_f = jnp.concatenate([src, zpad]).reshape(-1, CHUNK)
    sdst_f = jnp.concatenate([dst, npad]).reshape(-1, CHUNK)
    gsrc_r = jnp.concatenate([dst, zpad]).reshape(-1, CHUNK)
    sdst_r = jnp.concatenate([src, npad]).reshape(-1, CHUNK)

    flag = jnp.zeros((N, 1), x.dtype).at[mask_idx].set(1.0)
    x_m = x.at[mask_idx].set(0.0)

    p = params
    h = _tc_encode(x_m, flag, p["enc_in_W"][:IN_DIM],
                   _r(p["enc_in_W"][IN_DIM]), _r(p["enc_in_b"]))
    for lp in p["enc_layers"]:
        h = _down_up(h, gsrc_f, sdst_f, gsrc_r, sdst_r, lp)

    z = jnp.take(h, root_index, axis=0)
    dec_bias = z @ p["dec_in_W"][IN_DIM + 1:] + _r(p["dec_in_b"])
    hd = _tc_encode(x_m, flag, p["dec_in_W"][:IN_DIM],
                    _r(p["dec_in_W"][IN_DIM]), dec_bias)
    for lp in p["dec_layers"]:
        hd = _down_up(hd, gsrc_f, sdst_f, gsrc_r, sdst_r, lp)

    return _tc_out(hd, p["out_W1"], _r(p["out_b1"]), _r(p["out_g"]),
                   _r(p["out_bt"]), p["out_W2"], _r(p["out_b2"]))
